# baseline (speedup 1.0000x reference)
import functools

import jax
import jax.numpy as jnp
from jax import lax
from jax.experimental import pallas as pl
from jax.experimental.pallas import tpu as pltpu

_F32 = jnp.float32


def _l2norm(h):
    sumsq = jnp.sum(h * h, axis=-1, keepdims=True)
    return h * lax.rsqrt(jnp.maximum(sumsq, 1e-24))


def _fused_kernel(nb, num_snapshots, dim_inner,
                  x_ref, af_ref, as_lo_ref, as_hi_ref,
                  wpre_ref, bpre_ref, wmp0_ref, bmp0_ref,
                  wmp1_ref, b1_ref, whead_ref, bhead_ref,
                  o_ref, t_ref, acc_ref, w1_ref):
    i = pl.program_id(0)
    S, D = num_snapshots, dim_inner

    @pl.when(i == 0)
    def _init():
        h = jnp.dot(x_ref[...], wpre_ref[...],
                    preferred_element_type=_F32) + bpre_ref[...]
        h = jnp.maximum(h, 0.0)
        t_ref[...] = jnp.dot(h, wmp0_ref[...], preferred_element_type=_F32)
        w1_ref[...] = jnp.concatenate(
            [wmp1_ref[s] for s in range(S)], axis=1)
        acc_ref[...] = jnp.zeros_like(acc_ref)

    h1 = jnp.dot(af_ref[...], t_ref[...],
                 preferred_element_type=_F32) + bmp0_ref[...]
    h1 = _l2norm(jnp.maximum(h1, 0.0))

    u = jnp.dot(h1, w1_ref[...], preferred_element_type=_F32)
    half = S // 2
    partial = jnp.dot(as_lo_ref[0], u[:, 0:D], preferred_element_type=_F32)
    for s in range(1, half):
        partial = partial + jnp.dot(as_lo_ref[s], u[:, s * D:(s + 1) * D],
                                    preferred_element_type=_F32)
    for s in range(half, S):
        partial = partial + jnp.dot(as_hi_ref[s - half],
                                    u[:, s * D:(s + 1) * D],
                                    preferred_element_type=_F32)
    acc_ref[...] += partial

    @pl.when(i == nb - 1)
    def _finish():
        bsum = jnp.sum(b1_ref[...], axis=0)
        hf = _l2norm(jnp.maximum(acc_ref[...] + bsum, 0.0))
        out = jnp.dot(hf, whead_ref[...],
                      preferred_element_type=_F32) + bhead_ref[...]
        o_ref[...] = out.astype(o_ref.dtype)


def kernel(x, adj_full, adj_snapshots,
           w_pre, b_pre, w_mp0, b_mp0, w_mp1, b_mp1, w_head, b_head):
    N, dim_in = x.shape
    S = adj_snapshots.shape[0]
    dim_inner = w_pre.shape[1]
    dim_out = w_head.shape[1]

    blk = 256 if N % 256 == 0 and N > 256 else N
    nb = N // blk

    bpre = b_pre.reshape(1, dim_inner)
    bmp0 = b_mp0.reshape(1, dim_inner)
    b1 = b_mp1.reshape(S, 1, dim_inner)
    bhead = b_head.reshape(1, dim_out)

    return pl.pallas_call(
        functools.partial(_fused_kernel, nb, S, dim_inner),
        out_shape=jax.ShapeDtypeStruct((N, dim_out), x.dtype),
        grid=(nb,),
        in_specs=[
            pl.BlockSpec((N, dim_in), lambda i: (0, 0)),
            pl.BlockSpec((blk, N), lambda i: (i, 0)),
            pl.BlockSpec((S // 2, N, blk), lambda i: (0, 0, i)),
            pl.BlockSpec((S // 2, N, blk), lambda i: (1, 0, i)),
            pl.BlockSpec((dim_in, dim_inner), lambda i: (0, 0)),
            pl.BlockSpec((1, dim_inner), lambda i: (0, 0)),
            pl.BlockSpec((dim_inner, dim_inner), lambda i: (0, 0)),
            pl.BlockSpec((1, dim_inner), lambda i: (0, 0)),
            pl.BlockSpec((S, dim_inner, dim_inner), lambda i: (0, 0, 0)),
            pl.BlockSpec((S, 1, dim_inner), lambda i: (0, 0, 0)),
            pl.BlockSpec((dim_inner, dim_out), lambda i: (0, 0)),
            pl.BlockSpec((1, dim_out), lambda i: (0, 0)),
        ],
        out_specs=pl.BlockSpec((N, dim_out), lambda i: (0, 0)),
        scratch_shapes=[
            pltpu.VMEM((N, dim_inner), _F32),
            pltpu.VMEM((N, dim_inner), _F32),
            pltpu.VMEM((dim_inner, S * dim_inner), _F32),
        ],
        compiler_params=pltpu.CompilerParams(
            dimension_semantics=("arbitrary",)),
    )(x, adj_full, adj_snapshots, adj_snapshots,
      w_pre, bpre, w_mp0, bmp0, w_mp1, b1, w_head, bhead)


# speedup vs baseline: 1.0221x; 1.0221x over previous
import functools

import jax
import jax.numpy as jnp
from jax import lax
from jax.experimental import pallas as pl
from jax.experimental.pallas import tpu as pltpu

_F32 = jnp.float32


def _l2norm(h):
    sumsq = jnp.sum(h * h, axis=-1, keepdims=True)
    return h * lax.rsqrt(jnp.maximum(sumsq, 1e-24))


def _fused_kernel(nb, num_snapshots, dim_inner,
                  x_ref, af_ref, as_lo_ref, as_hi_ref,
                  wpre_ref, bpre_ref, wmp0_ref, bmp0_ref,
                  wmp1_ref, b1_ref, whead_ref, bhead_ref,
                  o_ref, t_ref, acc_ref, w1_ref):
    i = pl.program_id(0)
    S, D = num_snapshots, dim_inner

    @pl.when(i == 0)
    def _init():
        h = jnp.dot(x_ref[...], wpre_ref[...],
                    preferred_element_type=_F32) + bpre_ref[...]
        h = jnp.maximum(h, 0.0)
        t_ref[...] = jnp.dot(h, wmp0_ref[...], preferred_element_type=_F32)
        w1_ref[...] = jnp.concatenate(
            [wmp1_ref[s] for s in range(S)], axis=1)
        acc_ref[...] = jnp.zeros_like(acc_ref)

    h1 = jnp.dot(af_ref[...], t_ref[...],
                 preferred_element_type=_F32) + bmp0_ref[...]
    h1 = _l2norm(jnp.maximum(h1, 0.0))

    u = jnp.dot(h1, w1_ref[...], preferred_element_type=_F32)
    half = S // 2
    partial = jnp.dot(as_lo_ref[0], u[:, 0:D], preferred_element_type=_F32)
    for s in range(1, half):
        partial = partial + jnp.dot(as_lo_ref[s], u[:, s * D:(s + 1) * D],
                                    preferred_element_type=_F32)
    for s in range(half, S):
        partial = partial + jnp.dot(as_hi_ref[s - half],
                                    u[:, s * D:(s + 1) * D],
                                    preferred_element_type=_F32)
    acc_ref[...] += partial

    @pl.when(i == nb - 1)
    def _finish():
        bsum = jnp.sum(b1_ref[...], axis=0)
        hf = _l2norm(jnp.maximum(acc_ref[...] + bsum, 0.0))
        out = jnp.dot(hf, whead_ref[...],
                      preferred_element_type=_F32) + bhead_ref[...]
        o_ref[...] = out.astype(o_ref.dtype)


def kernel(x, adj_full, adj_snapshots,
           w_pre, b_pre, w_mp0, b_mp0, w_mp1, b_mp1, w_head, b_head):
    N, dim_in = x.shape
    S = adj_snapshots.shape[0]
    dim_inner = w_pre.shape[1]
    dim_out = w_head.shape[1]

    blk = 512 if N % 512 == 0 and N > 512 else N
    nb = N // blk

    bpre = b_pre.reshape(1, dim_inner)
    bmp0 = b_mp0.reshape(1, dim_inner)
    b1 = b_mp1.reshape(S, 1, dim_inner)
    bhead = b_head.reshape(1, dim_out)

    return pl.pallas_call(
        functools.partial(_fused_kernel, nb, S, dim_inner),
        out_shape=jax.ShapeDtypeStruct((N, dim_out), x.dtype),
        grid=(nb,),
        in_specs=[
            pl.BlockSpec((N, dim_in), lambda i: (0, 0)),
            pl.BlockSpec((blk, N), lambda i: (i, 0)),
            pl.BlockSpec((S // 2, N, blk), lambda i: (0, 0, i)),
            pl.BlockSpec((S // 2, N, blk), lambda i: (1, 0, i)),
            pl.BlockSpec((dim_in, dim_inner), lambda i: (0, 0)),
            pl.BlockSpec((1, dim_inner), lambda i: (0, 0)),
            pl.BlockSpec((dim_inner, dim_inner), lambda i: (0, 0)),
            pl.BlockSpec((1, dim_inner), lambda i: (0, 0)),
            pl.BlockSpec((S, dim_inner, dim_inner), lambda i: (0, 0, 0)),
            pl.BlockSpec((S, 1, dim_inner), lambda i: (0, 0, 0)),
            pl.BlockSpec((dim_inner, dim_out), lambda i: (0, 0)),
            pl.BlockSpec((1, dim_out), lambda i: (0, 0)),
        ],
        out_specs=pl.BlockSpec((N, dim_out), lambda i: (0, 0)),
        scratch_shapes=[
            pltpu.VMEM((N, dim_inner), _F32),
            pltpu.VMEM((N, dim_inner), _F32),
            pltpu.VMEM((dim_inner, S * dim_inner), _F32),
        ],
        compiler_params=pltpu.CompilerParams(
            dimension_semantics=("arbitrary",)),
    )(x, adj_full, adj_snapshots, adj_snapshots,
      w_pre, bpre, w_mp0, bmp0, w_mp1, b1, w_head, bhead)
